# Initial kernel scaffold; baseline (speedup 1.0000x reference)
import jax
import jax.numpy as jnp
from jax.experimental import pallas as pl


def kernel(x, edge_index, edge_attr, batch, Wf0, bf0, Ws0, bs0, g0, be0, Wf1, bf1, Ws1, bs1, g1, be1, W1, b1, W2, b2, W3, b3):
    raise NotImplementedError("write your pallas kernel here")


# speedup vs baseline: 1.9246x; 1.9246x over previous
import functools

import jax
import jax.numpy as jnp
from jax import lax
from jax.experimental import pallas as pl
from jax.experimental.pallas import tpu as pltpu
from jax.experimental.pallas import tpu_sc as plsc

N = 100000
E = 1600000
G = 128

NC = 2
NS = 16
K = 80
EPT = E // NS
NCHUNK = EPT // K
ROWS_PT = 6256
ROWS_LAST = N - 15 * ROWS_PT

_MESH = plsc.VectorSubcoreMesh(
    core_axis_name="c", subcore_axis_name="s", num_cores=NC, num_subcores=NS)

_SC_SCRATCH = [
    pltpu.VMEM_SHARED((N, 16), jnp.float32),
    pltpu.VMEM((9, 32), jnp.float32),
] + 2 * [
    pltpu.VMEM((K,), jnp.int32),
    pltpu.VMEM((K,), jnp.int32),
    pltpu.VMEM((K * 8,), jnp.float32),
    pltpu.VMEM((K, 32), jnp.float32),
    pltpu.VMEM((K, 32), jnp.float32),
] + [
    pltpu.VMEM((K, 16), jnp.float32),
] + 4 * [pltpu.SemaphoreType.DMA]


def _softplus16(x):
    t = jnp.exp(-jnp.abs(x))
    w = t / (t + 2.0)
    w2 = w * w
    p = w2 * (1.0 / 13.0) + (1.0 / 11.0)
    p = p * w2 + (1.0 / 9.0)
    p = p * w2 + (1.0 / 7.0)
    p = p * w2 + (1.0 / 5.0)
    p = p * w2 + (1.0 / 3.0)
    p = p * w2 + 1.0
    return jnp.maximum(x, 0.0) + 2.0 * w * p


def _sigmoid16(x):
    return 1.0 / (1.0 + jnp.exp(-x))


@functools.partial(
    pl.kernel,
    out_type=jax.ShapeDtypeStruct((NC, N, 16), jnp.float32),
    mesh=_MESH,
    scratch_types=_SC_SCRATCH,
    compiler_params=pltpu.CompilerParams(use_tc_tiling_on_sc=False),
)
def _sc_edge(p_hbm, q_hbm, dst_hbm, src_hbm, ea_hbm, wcb_hbm, zeros_hbm,
             out_hbm, agg, wcb_v,
             dA, sA, eA, pA, qA,
             dB, sB, eB, pB, qB,
             mbuf, semiA, semiB, semgA, semgB):
    c = lax.axis_index("c")
    s = lax.axis_index("s")
    base_e = s * EPT
    r0 = s * ROWS_PT

    @pl.when(s < 15)
    def _():
        pltpu.sync_copy(zeros_hbm.at[pl.ds(r0, ROWS_PT)],
                        agg.at[pl.ds(r0, ROWS_PT)])

    @pl.when(s == 15)
    def _():
        pltpu.sync_copy(zeros_hbm.at[pl.ds(15 * ROWS_PT, ROWS_LAST)],
                        agg.at[pl.ds(15 * ROWS_PT, ROWS_LAST)])

    pltpu.sync_copy(wcb_hbm.at[c], wcb_v)
    plsc.subcore_barrier()

    wcf = [wcb_v[k, 0:16] for k in range(8)]
    wcs = [wcb_v[k, 16:32] for k in range(8)]
    bf = wcb_v[8, 0:16]
    bs = wcb_v[8, 16:32]

    bufs = ((dA, sA, eA, pA, qA, semiA, semgA),
            (dB, sB, eB, pB, qB, semiB, semgB))

    def start_idx(i, b):
        d, sr, ea, _, _, si, _ = bufs[b]
        off = base_e + i * K
        pltpu.async_copy(dst_hbm.at[pl.ds(off, K)], d, si)
        pltpu.async_copy(src_hbm.at[pl.ds(off, K)], sr, si)
        pltpu.async_copy(ea_hbm.at[pl.ds(off * 8, K * 8)], ea, si)

    def wait_idx(b):
        d, sr, ea, _, _, si, _ = bufs[b]
        pltpu.make_async_copy(dst_hbm.at[pl.ds(0, K)], d, si).wait()
        pltpu.make_async_copy(src_hbm.at[pl.ds(0, K)], sr, si).wait()
        pltpu.make_async_copy(ea_hbm.at[pl.ds(0, K * 8)], ea, si).wait()

    def start_gather(b):
        d, sr, _, p, q, _, sg = bufs[b]
        pltpu.async_copy(p_hbm.at[c].at[d], p, sg)
        pltpu.async_copy(q_hbm.at[c].at[sr], q, sg)

    def wait_gather(b):
        d, sr, _, p, q, _, sg = bufs[b]
        pltpu.make_async_copy(p_hbm.at[c].at[d], p, sg).wait()
        pltpu.make_async_copy(q_hbm.at[c].at[sr], q, sg).wait()

    def compute(b):
        _, _, ea, p, q, _, _ = bufs[b]

        def edge(e, ev, eoff):
            af = bf
            asv = bs
            for k in range(8):
                sv = ev[eoff + k]
                af = af + sv * wcf[k]
                asv = asv + sv * wcs[k]
            lf = af + p[e, pl.ds(0, 16)] + q[e, pl.ds(0, 16)]
            ls = asv + p[e, pl.ds(16, 16)] + q[e, pl.ds(16, 16)]
            mbuf[e, pl.ds(0, 16)] = _sigmoid16(lf) * _softplus16(ls)

        def body(e2, carry):
            ev = ea[pl.ds(e2 * 16, 16)]
            edge(2 * e2, ev, 0)
            edge(2 * e2 + 1, ev, 8)
            return carry

        lax.fori_loop(0, K // 2, body, 0)

    start_idx(0, 0)
    wait_idx(0)
    start_gather(0)
    start_idx(1, 1)

    def outer(i2, carry):
        for half in range(2):
            i = 2 * i2 + half
            b = half
            nb = 1 - half
            wait_gather(b)

            @pl.when(i + 1 < NCHUNK)
            def _():
                wait_idx(nb)
                start_gather(nb)

            compute(b)
            pltpu.sync_copy(mbuf, agg.at[bufs[b][0]], add=True)

            @pl.when(i + 2 < NCHUNK)
            def _():
                start_idx(i + 2, b)

        return carry

    lax.fori_loop(0, NCHUNK // 2, outer, 0)

    plsc.subcore_barrier()

    @pl.when(s < 15)
    def _():
        pltpu.sync_copy(agg.at[pl.ds(r0, ROWS_PT)],
                        out_hbm.at[c, pl.ds(r0, ROWS_PT)])

    @pl.when(s == 15)
    def _():
        pltpu.sync_copy(agg.at[pl.ds(15 * ROWS_PT, ROWS_LAST)],
                        out_hbm.at[c, pl.ds(15 * ROWS_PT, ROWS_LAST)])


BN = 2000
NB = N // BN


def _split_pq(pq):
    p = jnp.stack([pq[:, 0:32], pq[:, 32:64]])
    q = jnp.stack([pq[:, 64:96], pq[:, 96:128]])
    return p, q


def _proj_body(x_ref, w_ref, p_ref, q_ref):
    pq = jnp.dot(x_ref[...], w_ref[...], preferred_element_type=jnp.float32)
    p_ref[...], q_ref[...] = _split_pq(pq)


def _proj(x, wpq):
    return pl.pallas_call(
        _proj_body,
        grid=(NB,),
        in_specs=[
            pl.BlockSpec((BN, 20), lambda i: (i, 0)),
            pl.BlockSpec((20, 128), lambda i: (0, 0)),
        ],
        out_specs=[
            pl.BlockSpec((NC, BN, 32), lambda i: (0, i, 0)),
            pl.BlockSpec((NC, BN, 32), lambda i: (0, i, 0)),
        ],
        out_shape=[
            jax.ShapeDtypeStruct((NC, N, 32), jnp.float32),
            jax.ShapeDtypeStruct((NC, N, 32), jnp.float32),
        ],
    )(x, wpq)


def _merge_agg(a0, a1):
    return jnp.concatenate([a0, a1[:, 12:16]], axis=1)


def _stats_body(x_ref, a0_ref, a1_ref, y_ref, st_ref):
    i = pl.program_id(0)
    y = x_ref[...] + _merge_agg(a0_ref[...], a1_ref[...])
    y_ref[...] = y

    @pl.when(i == 0)
    def _():
        st_ref[...] = jnp.zeros_like(st_ref)

    st_ref[...] += jnp.stack([jnp.sum(y, 0), jnp.sum(y * y, 0)])


def _stats(x, a0, a1):
    return pl.pallas_call(
        _stats_body,
        grid=(NB,),
        in_specs=[
            pl.BlockSpec((BN, 20), lambda i: (i, 0)),
            pl.BlockSpec((BN, 16), lambda i: (i, 0)),
            pl.BlockSpec((BN, 16), lambda i: (i, 0)),
        ],
        out_specs=[
            pl.BlockSpec((BN, 20), lambda i: (i, 0)),
            pl.BlockSpec((2, 20), lambda i: (0, 0)),
        ],
        out_shape=[
            jax.ShapeDtypeStruct((N, 20), jnp.float32),
            jax.ShapeDtypeStruct((2, 20), jnp.float32),
        ],
    )(x, a0, a1)


def _bn_apply(y, st_ref, gb_ref):
    mu = st_ref[0:1, :] * (1.0 / N)
    var = st_ref[1:2, :] * (1.0 / N) - mu * mu
    scale = gb_ref[0:1, :] * jax.lax.rsqrt(var + 1e-5)
    return jnp.tanh((y - mu) * scale + gb_ref[1:2, :])


def _apply_proj_body(y_ref, st_ref, gb_ref, w_ref, x1_ref, p_ref, q_ref):
    xn = _bn_apply(y_ref[...], st_ref, gb_ref)
    x1_ref[...] = xn
    pq = jnp.dot(xn, w_ref[...], preferred_element_type=jnp.float32)
    p_ref[...], q_ref[...] = _split_pq(pq)


def _apply_proj(y, st, gb, wpq):
    return pl.pallas_call(
        _apply_proj_body,
        grid=(NB,),
        in_specs=[
            pl.BlockSpec((BN, 20), lambda i: (i, 0)),
            pl.BlockSpec((2, 20), lambda i: (0, 0)),
            pl.BlockSpec((2, 20), lambda i: (0, 0)),
            pl.BlockSpec((20, 128), lambda i: (0, 0)),
        ],
        out_specs=[
            pl.BlockSpec((BN, 20), lambda i: (i, 0)),
            pl.BlockSpec((NC, BN, 32), lambda i: (0, i, 0)),
            pl.BlockSpec((NC, BN, 32), lambda i: (0, i, 0)),
        ],
        out_shape=[
            jax.ShapeDtypeStruct((N, 20), jnp.float32),
            jax.ShapeDtypeStruct((NC, N, 32), jnp.float32),
            jax.ShapeDtypeStruct((NC, N, 32), jnp.float32),
        ],
    )(y, st, gb, wpq)


def _apply_pool_body(y_ref, st_ref, gb_ref, batch_ref, pooled_ref):
    i = pl.program_id(0)
    xn = _bn_apply(y_ref[...], st_ref, gb_ref)
    b = batch_ref[0, 0, :]
    oh = (b[:, None] == lax.broadcasted_iota(jnp.int32, (1, G), 1)
          ).astype(jnp.float32)

    @pl.when(i == 0)
    def _():
        pooled_ref[...] = jnp.zeros_like(pooled_ref)

    pooled_ref[...] += lax.dot_general(
        oh, xn, (((0,), (0,)), ((), ())), preferred_element_type=jnp.float32)


def _apply_pool(y, st, gb, batch3):
    return pl.pallas_call(
        _apply_pool_body,
        grid=(NB,),
        in_specs=[
            pl.BlockSpec((BN, 20), lambda i: (i, 0)),
            pl.BlockSpec((2, 20), lambda i: (0, 0)),
            pl.BlockSpec((2, 20), lambda i: (0, 0)),
            pl.BlockSpec((1, 1, BN), lambda i: (i, 0, 0)),
        ],
        out_specs=pl.BlockSpec((G, 20), lambda i: (0, 0)),
        out_shape=jax.ShapeDtypeStruct((G, 20), jnp.float32),
    )(y, st, gb, batch3)


def _mlp_body(pooled_ref, w1_ref, b1_ref, w2_ref, b2_ref, w3_ref, b3_ref,
              out_ref):
    h = jnp.tanh(jnp.dot(pooled_ref[...], w1_ref[...],
                         preferred_element_type=jnp.float32) + b1_ref[...])
    h = jnp.tanh(jnp.dot(h, w2_ref[...],
                         preferred_element_type=jnp.float32) + b2_ref[...])
    out_ref[...] = jnp.dot(h, w3_ref[...],
                           preferred_element_type=jnp.float32) + b3_ref[...]


def _mlp(pooled, W1, b1, W2, b2, W3, b3):
    return pl.pallas_call(
        _mlp_body,
        out_shape=jax.ShapeDtypeStruct((G, 1), jnp.float32),
    )(pooled, W1, b1[None], W2, b2[None], W3, b3[None])


def _win32(a, b, c):
    lo = 0 if c == 0 else 4
    return jnp.concatenate([a[:, lo:lo + 16], b[:, lo:lo + 16]], axis=1)


def _wpq(Wf, Ws):
    return jnp.concatenate(
        [_win32(Wf[0:20], Ws[0:20], 0), _win32(Wf[0:20], Ws[0:20], 1),
         _win32(Wf[20:40], Ws[20:40], 0), _win32(Wf[20:40], Ws[20:40], 1)],
        axis=1)


def _wcb(Wf, Ws, bf, bs):
    out = []
    for c in range(NC):
        wcrows = _win32(Wf[40:48], Ws[40:48], c)
        brow = _win32(bf[None], bs[None], c)
        out.append(jnp.concatenate([wcrows, brow], axis=0))
    return jnp.stack(out)


def kernel(x, edge_index, edge_attr, batch, Wf0, bf0, Ws0, bs0, g0, be0,
           Wf1, bf1, Ws1, bs1, g1, be1, W1, b1, W2, b2, W3, b3):
    src = edge_index[0]
    dst = edge_index[1]
    zeros = jnp.zeros((N, 16), jnp.float32)
    batch3 = batch.reshape(NB, 1, BN)
    eaflat = edge_attr.reshape(E * 8)

    P0, Q0 = _proj(x, _wpq(Wf0, Ws0))
    agg0 = _sc_edge(P0, Q0, dst, src, eaflat, _wcb(Wf0, Ws0, bf0, bs0),
                    zeros)
    y0, st0 = _stats(x, agg0[0], agg0[1])
    x1, P1, Q1 = _apply_proj(y0, st0, jnp.stack([g0, be0]), _wpq(Wf1, Ws1))
    agg1 = _sc_edge(P1, Q1, dst, src, eaflat, _wcb(Wf1, Ws1, bf1, bs1),
                    zeros)
    y1, st1 = _stats(x1, agg1[0], agg1[1])
    pooled = _apply_pool(y1, st1, jnp.stack([g1, be1]), batch3)
    return _mlp(pooled, W1, b1, W2, b2, W3, b3)
